# baseline (speedup 1.0000x reference)
import functools

import jax
import jax.numpy as jnp
from jax import lax
from jax.experimental import pallas as pl
from jax.experimental.pallas import tpu as pltpu
from jax.experimental.pallas import tpu_sc as plsc

_R = 256
_E = 4096
_W = 512


def _argmin_body(x_ref, embt_ref, s_ref, c_ref, idx_ref, loss_ref,
                 accv_ref, acci_ref, chosen_ref):
    i = pl.program_id(0)
    j = pl.program_id(1)
    nj = pl.num_programs(1)

    @pl.when(jnp.logical_and(i == 0, j == 0))
    def _():
        loss_ref[0, 0] = 0.0

    @pl.when(j == 0)
    def _():
        accv_ref[...] = jnp.full_like(accv_ref, jnp.inf)
        acci_ref[...] = jnp.zeros_like(acci_ref)
        chosen_ref[...] = jnp.full_like(chosen_ref, jnp.inf)

    xb = (2.0 * x_ref[...]).astype(jnp.bfloat16).astype(jnp.float32)
    s = s_ref[...]
    r = s.shape[0]
    cm = jnp.full((r, 1), jnp.inf, jnp.float32)
    ci = jnp.full((r, 1), jnp.int32(2147483647), jnp.int32)
    for k in range(_E // _W):
        e = embt_ref[:, k * _W:(k + 1) * _W]
        m = jnp.dot(xb, e, preferred_element_type=jnp.float32)
        d = (s - m) + c_ref[:, k * _W:(k + 1) * _W]
        wm = jnp.min(d, axis=1, keepdims=True)
        ii = lax.broadcasted_iota(jnp.int32, d.shape, 1) + (j * _E + k * _W)
        wi = jnp.min(jnp.where(d == wm, ii, jnp.int32(2147483647)),
                     axis=1, keepdims=True)
        tk = jnp.logical_or(cm < wm, jnp.logical_and(cm == wm, ci < wi))
        cm = jnp.where(tk, cm, wm)
        ci = jnp.where(tk, ci, wi)
    accv = accv_ref[...]
    acci = acci_ref[...]
    keep = jnp.logical_or(accv < cm,
                          jnp.logical_and(accv == cm, acci < ci))
    cmr = cm.astype(jnp.bfloat16).astype(jnp.float32)
    accv_ref[...] = jnp.where(keep, accv, cmr)
    acci_ref[...] = jnp.where(keep, acci, ci)
    chosen_ref[...] = jnp.where(keep, chosen_ref[...], cm)

    @pl.when(j == nj - 1)
    def _():
        idx_ref[...] = acci_ref[...][:, 0]
        loss_ref[0, 0] += jnp.sum(chosen_ref[...])


def _tc_argmin(flat, embt, s, c):
    n_rows, _ = flat.shape
    n_emb = embt.shape[1]
    grid = (n_rows // _R, n_emb // _E)
    idx, loss = pl.pallas_call(
        _argmin_body,
        grid=grid,
        in_specs=[
            pl.BlockSpec((_R, flat.shape[1]), lambda i, j: (i, 0)),
            pl.BlockSpec((embt.shape[0], _E), lambda i, j: (0, j)),
            pl.BlockSpec((_R, 1), lambda i, j: (i, 0)),
            pl.BlockSpec((1, _E), lambda i, j: (0, j)),
        ],
        out_specs=[
            pl.BlockSpec((_R,), lambda i, j: (i,)),
            pl.BlockSpec(block_shape=(1, 1), index_map=lambda i, j: (0, 0),
                         memory_space=pltpu.SMEM),
        ],
        out_shape=[
            jax.ShapeDtypeStruct((n_rows,), jnp.int32),
            jax.ShapeDtypeStruct((1, 1), jnp.float32),
        ],
        scratch_shapes=[
            pltpu.VMEM((_R, 1), jnp.float32),
            pltpu.VMEM((_R, 1), jnp.int32),
            pltpu.VMEM((_R, 1), jnp.float32),
        ],
    )(flat, embt, s, c)
    return idx, loss


@functools.cache
def _make_sc_gather(n_emb, d, b):
    info = plsc.get_sparse_core_info()
    nc, ns = info.num_cores, info.num_subcores
    nw = nc * ns
    b_per_w = b // nw
    mesh = plsc.VectorSubcoreMesh(core_axis_name="c", subcore_axis_name="s")

    @functools.partial(
        pl.kernel,
        mesh=mesh,
        out_type=jax.ShapeDtypeStruct((b, d), jnp.float32),
        scratch_types=[
            pltpu.VMEM((b_per_w,), jnp.int32),
            pltpu.VMEM((b_per_w, d), jnp.float32),
            pltpu.SemaphoreType.DMA,
        ],
    )
    def gather_kernel(table_hbm, idx_hbm, out_hbm, idx_v, rows_v, sem):
        wid = lax.axis_index("s") * nc + lax.axis_index("c")
        base = wid * b_per_w
        pltpu.sync_copy(idx_hbm.at[pl.ds(base, b_per_w)], idx_v)
        pltpu.async_copy(table_hbm.at[idx_v], rows_v, sem).wait()
        pltpu.sync_copy(rows_v, out_hbm.at[pl.ds(base, b_per_w)])

    return gather_kernel


def kernel(x, emb):
    flat = x.reshape(-1, x.shape[-1])
    s = jnp.sum(flat ** 2, axis=1, keepdims=True)
    c = jnp.sum(emb ** 2, axis=1)[None, :]
    embt = emb.T
    idx, loss = _tc_argmin(flat, embt, s, c)
    emb_pad = jnp.pad(emb, ((0, 0), (0, 128 - emb.shape[1])))
    quant = _make_sc_gather(emb.shape[0], 128, flat.shape[0])(
        emb_pad, idx)[:, : emb.shape[1]]
    out = (flat + (quant - flat)).reshape(x.shape)
    loss_val = loss[0, 0] * (1.0 / flat.size)
    return out, idx.reshape(x.shape[:2]), loss_val


# speedup vs baseline: 1.3227x; 1.3227x over previous
import functools

import jax
import jax.numpy as jnp
from jax import lax
from jax.experimental import pallas as pl
from jax.experimental.pallas import tpu as pltpu
from jax.experimental.pallas import tpu_sc as plsc

_R = 512
_E = 4096


def _argmin_body(x_ref, embt_ref, s_ref, c_ref, idx_ref, loss_ref,
                 accv_ref, acci_ref, chosen_ref):
    i = pl.program_id(0)
    j = pl.program_id(1)
    nj = pl.num_programs(1)

    @pl.when(jnp.logical_and(i == 0, j == 0))
    def _():
        loss_ref[0, 0] = 0.0

    @pl.when(j == 0)
    def _():
        accv_ref[...] = jnp.full_like(accv_ref, jnp.inf)
        acci_ref[...] = jnp.zeros_like(acci_ref)
        chosen_ref[...] = jnp.full_like(chosen_ref, jnp.inf)

    xb = (2.0 * x_ref[...]).astype(jnp.bfloat16).astype(jnp.float32)
    e = embt_ref[...]
    m = jnp.dot(xb, e, preferred_element_type=jnp.float32)
    d = (s_ref[...] - m) + c_ref[...]
    cm = jnp.min(d, axis=1, keepdims=True)
    ii = lax.broadcasted_iota(jnp.int32, d.shape, 1) + j * _E
    ci = jnp.min(jnp.where(d == cm, ii, jnp.int32(2147483647)),
                 axis=1, keepdims=True)
    accv = accv_ref[...]
    acci = acci_ref[...]
    keep = jnp.logical_or(accv < cm,
                          jnp.logical_and(accv == cm, acci < ci))
    cmr = cm.astype(jnp.bfloat16).astype(jnp.float32)
    accv_ref[...] = jnp.where(keep, accv, cmr)
    acci_ref[...] = jnp.where(keep, acci, ci)
    chosen_ref[...] = jnp.where(keep, chosen_ref[...], cm)

    @pl.when(j == nj - 1)
    def _():
        idx_ref[...] = acci_ref[...][:, 0]
        loss_ref[0, 0] += jnp.sum(chosen_ref[...])


def _tc_argmin(flat, embt, s, c):
    n_rows, _ = flat.shape
    n_emb = embt.shape[1]
    grid = (n_rows // _R, n_emb // _E)
    idx, loss = pl.pallas_call(
        _argmin_body,
        grid=grid,
        in_specs=[
            pl.BlockSpec((_R, flat.shape[1]), lambda i, j: (i, 0)),
            pl.BlockSpec((embt.shape[0], _E), lambda i, j: (0, j)),
            pl.BlockSpec((_R, 1), lambda i, j: (i, 0)),
            pl.BlockSpec((1, _E), lambda i, j: (0, j)),
        ],
        out_specs=[
            pl.BlockSpec((_R,), lambda i, j: (i,)),
            pl.BlockSpec(block_shape=(1, 1), index_map=lambda i, j: (0, 0),
                         memory_space=pltpu.SMEM),
        ],
        out_shape=[
            jax.ShapeDtypeStruct((n_rows,), jnp.int32),
            jax.ShapeDtypeStruct((1, 1), jnp.float32),
        ],
        scratch_shapes=[
            pltpu.VMEM((_R, 1), jnp.float32),
            pltpu.VMEM((_R, 1), jnp.int32),
            pltpu.VMEM((_R, 1), jnp.float32),
        ],
    )(flat, embt, s, c)
    return idx, loss


@functools.cache
def _make_sc_gather(n_emb, d, b):
    info = plsc.get_sparse_core_info()
    nc, ns = info.num_cores, info.num_subcores
    nw = nc * ns
    b_per_w = b // nw
    mesh = plsc.VectorSubcoreMesh(core_axis_name="c", subcore_axis_name="s")

    @functools.partial(
        pl.kernel,
        mesh=mesh,
        out_type=jax.ShapeDtypeStruct((b, d), jnp.float32),
        scratch_types=[
            pltpu.VMEM((b_per_w,), jnp.int32),
            pltpu.VMEM((b_per_w, d), jnp.float32),
            pltpu.SemaphoreType.DMA,
        ],
    )
    def gather_kernel(table_hbm, idx_hbm, out_hbm, idx_v, rows_v, sem):
        wid = lax.axis_index("s") * nc + lax.axis_index("c")
        base = wid * b_per_w
        pltpu.sync_copy(idx_hbm.at[pl.ds(base, b_per_w)], idx_v)
        pltpu.async_copy(table_hbm.at[idx_v], rows_v, sem).wait()
        pltpu.sync_copy(rows_v, out_hbm.at[pl.ds(base, b_per_w)])

    return gather_kernel


def kernel(x, emb):
    flat = x.reshape(-1, x.shape[-1])
    s = jnp.sum(flat ** 2, axis=1, keepdims=True)
    c = jnp.sum(emb ** 2, axis=1)[None, :]
    embt = emb.T
    idx, loss = _tc_argmin(flat, embt, s, c)
    emb_pad = jnp.pad(emb, ((0, 0), (0, 128 - emb.shape[1])))
    quant = _make_sc_gather(emb.shape[0], 128, flat.shape[0])(
        emb_pad, idx)[:, : emb.shape[1]]
    out = (flat + (quant - flat)).reshape(x.shape)
    loss_val = loss[0, 0] * (1.0 / flat.size)
    return out, idx.reshape(x.shape[:2]), loss_val


# speedup vs baseline: 1.3712x; 1.0367x over previous
import functools

import jax
import jax.numpy as jnp
from jax import lax
from jax.experimental import pallas as pl
from jax.experimental.pallas import tpu as pltpu
from jax.experimental.pallas import tpu_sc as plsc

_R = 1024
_E = 4096


def _argmin_body(x_ref, embt_ref, s_ref, c_ref, idx_ref, loss_ref,
                 accv_ref, acci_ref, chosen_ref):
    i = pl.program_id(0)
    j = pl.program_id(1)
    nj = pl.num_programs(1)

    @pl.when(jnp.logical_and(i == 0, j == 0))
    def _():
        loss_ref[0, 0] = 0.0

    @pl.when(j == 0)
    def _():
        accv_ref[...] = jnp.full_like(accv_ref, jnp.inf)
        acci_ref[...] = jnp.zeros_like(acci_ref)
        chosen_ref[...] = jnp.full_like(chosen_ref, jnp.inf)

    xb = (2.0 * x_ref[...]).astype(jnp.bfloat16).astype(jnp.float32)
    e = embt_ref[...]
    m = jnp.dot(xb, e, preferred_element_type=jnp.float32)
    d = (s_ref[...] - m) + c_ref[...]
    cm = jnp.min(d, axis=1, keepdims=True)
    ii = lax.broadcasted_iota(jnp.int32, d.shape, 1) + j * _E
    ci = jnp.min(jnp.where(d == cm, ii, jnp.int32(2147483647)),
                 axis=1, keepdims=True)
    accv = accv_ref[...]
    acci = acci_ref[...]
    keep = jnp.logical_or(accv < cm,
                          jnp.logical_and(accv == cm, acci < ci))
    cmr = cm.astype(jnp.bfloat16).astype(jnp.float32)
    accv_ref[...] = jnp.where(keep, accv, cmr)
    acci_ref[...] = jnp.where(keep, acci, ci)
    chosen_ref[...] = jnp.where(keep, chosen_ref[...], cm)

    @pl.when(j == nj - 1)
    def _():
        idx_ref[...] = acci_ref[...][:, 0]
        loss_ref[0, 0] += jnp.sum(chosen_ref[...])


def _tc_argmin(flat, embt, s, c):
    n_rows, _ = flat.shape
    n_emb = embt.shape[1]
    grid = (n_rows // _R, n_emb // _E)
    idx, loss = pl.pallas_call(
        _argmin_body,
        grid=grid,
        in_specs=[
            pl.BlockSpec((_R, flat.shape[1]), lambda i, j: (i, 0)),
            pl.BlockSpec((embt.shape[0], _E), lambda i, j: (0, j)),
            pl.BlockSpec((_R, 1), lambda i, j: (i, 0)),
            pl.BlockSpec((1, _E), lambda i, j: (0, j)),
        ],
        out_specs=[
            pl.BlockSpec((_R,), lambda i, j: (i,)),
            pl.BlockSpec(block_shape=(1, 1), index_map=lambda i, j: (0, 0),
                         memory_space=pltpu.SMEM),
        ],
        out_shape=[
            jax.ShapeDtypeStruct((n_rows,), jnp.int32),
            jax.ShapeDtypeStruct((1, 1), jnp.float32),
        ],
        scratch_shapes=[
            pltpu.VMEM((_R, 1), jnp.float32),
            pltpu.VMEM((_R, 1), jnp.int32),
            pltpu.VMEM((_R, 1), jnp.float32),
        ],
    )(flat, embt, s, c)
    return idx, loss


@functools.cache
def _make_sc_gather(n_emb, d, b):
    info = plsc.get_sparse_core_info()
    nc, ns = info.num_cores, info.num_subcores
    nw = nc * ns
    b_per_w = b // nw
    mesh = plsc.VectorSubcoreMesh(core_axis_name="c", subcore_axis_name="s")

    @functools.partial(
        pl.kernel,
        mesh=mesh,
        out_type=jax.ShapeDtypeStruct((b, d), jnp.float32),
        scratch_types=[
            pltpu.VMEM((b_per_w,), jnp.int32),
            pltpu.VMEM((b_per_w, d), jnp.float32),
            pltpu.SemaphoreType.DMA,
        ],
    )
    def gather_kernel(table_hbm, idx_hbm, out_hbm, idx_v, rows_v, sem):
        wid = lax.axis_index("s") * nc + lax.axis_index("c")
        base = wid * b_per_w
        pltpu.sync_copy(idx_hbm.at[pl.ds(base, b_per_w)], idx_v)
        pltpu.async_copy(table_hbm.at[idx_v], rows_v, sem).wait()
        pltpu.sync_copy(rows_v, out_hbm.at[pl.ds(base, b_per_w)])

    return gather_kernel


def kernel(x, emb):
    flat = x.reshape(-1, x.shape[-1])
    s = jnp.sum(flat ** 2, axis=1, keepdims=True)
    c = jnp.sum(emb ** 2, axis=1)[None, :]
    embt = emb.T
    idx, loss = _tc_argmin(flat, embt, s, c)
    emb_pad = jnp.pad(emb, ((0, 0), (0, 128 - emb.shape[1])))
    quant = _make_sc_gather(emb.shape[0], 128, flat.shape[0])(
        emb_pad, idx)[:, : emb.shape[1]]
    out = (flat + (quant - flat)).reshape(x.shape)
    loss_val = loss[0, 0] * (1.0 / flat.size)
    return out, idx.reshape(x.shape[:2]), loss_val


# speedup vs baseline: 1.4001x; 1.0210x over previous
import functools

import jax
import jax.numpy as jnp
from jax import lax
from jax.experimental import pallas as pl
from jax.experimental.pallas import tpu as pltpu
from jax.experimental.pallas import tpu_sc as plsc

_R = 2048
_E = 4096


def _argmin_body(x_ref, embt_ref, s_ref, c_ref, idx_ref, loss_ref,
                 accv_ref, acci_ref, chosen_ref):
    i = pl.program_id(0)
    j = pl.program_id(1)
    nj = pl.num_programs(1)

    @pl.when(jnp.logical_and(i == 0, j == 0))
    def _():
        loss_ref[0, 0] = 0.0

    @pl.when(j == 0)
    def _():
        accv_ref[...] = jnp.full_like(accv_ref, jnp.inf)
        acci_ref[...] = jnp.zeros_like(acci_ref)
        chosen_ref[...] = jnp.full_like(chosen_ref, jnp.inf)

    xb = (2.0 * x_ref[...]).astype(jnp.bfloat16).astype(jnp.float32)
    e = embt_ref[...]
    m = jnp.dot(xb, e, preferred_element_type=jnp.float32)
    d = (s_ref[...] - m) + c_ref[...]
    cm = jnp.min(d, axis=1, keepdims=True)
    ii = lax.broadcasted_iota(jnp.int32, d.shape, 1) + j * _E
    ci = jnp.min(jnp.where(d == cm, ii, jnp.int32(2147483647)),
                 axis=1, keepdims=True)
    accv = accv_ref[...]
    acci = acci_ref[...]
    keep = jnp.logical_or(accv < cm,
                          jnp.logical_and(accv == cm, acci < ci))
    cmr = cm.astype(jnp.bfloat16).astype(jnp.float32)
    accv_ref[...] = jnp.where(keep, accv, cmr)
    acci_ref[...] = jnp.where(keep, acci, ci)
    chosen_ref[...] = jnp.where(keep, chosen_ref[...], cm)

    @pl.when(j == nj - 1)
    def _():
        idx_ref[...] = acci_ref[...][:, 0]
        loss_ref[0, 0] += jnp.sum(chosen_ref[...])


def _tc_argmin(flat, embt, s, c):
    n_rows, _ = flat.shape
    n_emb = embt.shape[1]
    grid = (n_rows // _R, n_emb // _E)
    idx, loss = pl.pallas_call(
        _argmin_body,
        grid=grid,
        in_specs=[
            pl.BlockSpec((_R, flat.shape[1]), lambda i, j: (i, 0)),
            pl.BlockSpec((embt.shape[0], _E), lambda i, j: (0, j)),
            pl.BlockSpec((_R, 1), lambda i, j: (i, 0)),
            pl.BlockSpec((1, _E), lambda i, j: (0, j)),
        ],
        out_specs=[
            pl.BlockSpec((_R,), lambda i, j: (i,)),
            pl.BlockSpec(block_shape=(1, 1), index_map=lambda i, j: (0, 0),
                         memory_space=pltpu.SMEM),
        ],
        out_shape=[
            jax.ShapeDtypeStruct((n_rows,), jnp.int32),
            jax.ShapeDtypeStruct((1, 1), jnp.float32),
        ],
        scratch_shapes=[
            pltpu.VMEM((_R, 1), jnp.float32),
            pltpu.VMEM((_R, 1), jnp.int32),
            pltpu.VMEM((_R, 1), jnp.float32),
        ],
    )(flat, embt, s, c)
    return idx, loss


@functools.cache
def _make_sc_gather(n_emb, d, b):
    info = plsc.get_sparse_core_info()
    nc, ns = info.num_cores, info.num_subcores
    nw = nc * ns
    b_per_w = b // nw
    mesh = plsc.VectorSubcoreMesh(core_axis_name="c", subcore_axis_name="s")

    @functools.partial(
        pl.kernel,
        mesh=mesh,
        out_type=jax.ShapeDtypeStruct((b, d), jnp.float32),
        scratch_types=[
            pltpu.VMEM((b_per_w,), jnp.int32),
            pltpu.VMEM((b_per_w, d), jnp.float32),
            pltpu.SemaphoreType.DMA,
        ],
    )
    def gather_kernel(table_hbm, idx_hbm, out_hbm, idx_v, rows_v, sem):
        wid = lax.axis_index("s") * nc + lax.axis_index("c")
        base = wid * b_per_w
        pltpu.sync_copy(idx_hbm.at[pl.ds(base, b_per_w)], idx_v)
        pltpu.async_copy(table_hbm.at[idx_v], rows_v, sem).wait()
        pltpu.sync_copy(rows_v, out_hbm.at[pl.ds(base, b_per_w)])

    return gather_kernel


def kernel(x, emb):
    flat = x.reshape(-1, x.shape[-1])
    s = jnp.sum(flat ** 2, axis=1, keepdims=True)
    c = jnp.sum(emb ** 2, axis=1)[None, :]
    embt = emb.T
    idx, loss = _tc_argmin(flat, embt, s, c)
    emb_pad = jnp.pad(emb, ((0, 0), (0, 128 - emb.shape[1])))
    quant = _make_sc_gather(emb.shape[0], 128, flat.shape[0])(
        emb_pad, idx)[:, : emb.shape[1]]
    out = (flat + (quant - flat)).reshape(x.shape)
    loss_val = loss[0, 0] * (1.0 / flat.size)
    return out, idx.reshape(x.shape[:2]), loss_val
